# Initial kernel scaffold; baseline (speedup 1.0000x reference)
import jax
import jax.numpy as jnp
from jax.experimental import pallas as pl


def kernel(x, target):
    raise NotImplementedError("write your pallas kernel here")


# speedup vs baseline: 66.0345x; 66.0345x over previous
import functools

import jax
import jax.numpy as jnp
from jax import lax
from jax.experimental import pallas as pl
from jax.experimental.pallas import tpu as pltpu
from jax.experimental.pallas import tpu_sc as plsc

NUM_CLASSES = 21
NPIX = 4 * 512 * 512
SPATIAL = 512 * 512
NBINS = 1024
TBL = 2 * NBINS
LANES = 16
NCORES = 2
NSUB = 16
NW = NCORES * NSUB
PER_W = NPIX // NW
VECS = PER_W // LANES
UNROLL = 4
ROWS_PER_BATCH = SPATIAL // 128
ERR_BLOCK_ROWS = 256


def _index_body(x_ref, t_ref, o_ref):
    xb = x_ref[0]
    m = jnp.max(xb, axis=0, keepdims=True)
    ex = jnp.exp(xb - m)
    p = ex / jnp.sum(ex, axis=0, keepdims=True)
    tb = t_ref[0]
    cls = lax.broadcasted_iota(jnp.int32, (NUM_CLASSES, 1, 1), 0)
    fg = tb[None, :, :] == cls
    eabs = jnp.abs(fg.astype(jnp.float32) - p)
    bn = jnp.minimum((eabs * float(NBINS)).astype(jnp.int32),
                     jnp.int32(NBINS - 1))
    lane = lax.broadcasted_iota(jnp.int32,
                                (NUM_CLASSES, ERR_BLOCK_ROWS, 128), 2) & 15
    o_ref[0] = lane * TBL + jnp.where(fg, jnp.int32(NBINS), 0) + bn


def _sc_hist_body(idx_ref, out_ref, buf, table, red):
    core = lax.axis_index("c")
    sub = lax.axis_index("s")
    w = sub * NCORES + core
    ones = jnp.full((LANES,), 1.0, jnp.float32)
    zeros = jnp.zeros((LANES,), jnp.float32)

    def _zt(j, carry):
        table[pl.ds(j * 16, 16)] = zeros
        return carry
    lax.fori_loop(0, LANES * TBL // 16, _zt, 0)

    b = w // 8
    p8 = w % 8

    def _class(c, carry):
        off = (b * NUM_CLASSES + c) * SPATIAL + p8 * PER_W
        pltpu.sync_copy(idx_ref.at[pl.ds(off, PER_W)], buf)

        def _vec(i, carry2):
            for u in range(UNROLL):
                v = buf[pl.ds((i * UNROLL + u) * LANES, LANES)]
                plsc.addupdate_scatter(table, [v], ones)
            return carry2
        lax.fori_loop(0, VECS // UNROLL, _vec, 0)

        def _red(j, carry2):
            col = j * 16
            acc = table[pl.ds(col, 16)]
            table[pl.ds(col, 16)] = zeros
            for l in range(1, LANES):
                acc = acc + table[pl.ds(l * TBL + col, 16)]
                table[pl.ds(l * TBL + col, 16)] = zeros
            red[pl.ds(col, 16)] = acc
            return carry2
        lax.fori_loop(0, TBL // 16, _red, 0)

        pltpu.sync_copy(red, out_ref.at[c, w])
        return carry
    lax.fori_loop(0, NUM_CLASSES, _class, 0)


def _finalize_body(h_ref, o_ref):
    h = h_ref[...]
    h = jnp.sum(h, axis=1)
    n0 = h[:, :NBINS]
    n1 = h[:, NBINS:]
    n = n0 + n1

    def cumsum_last(v):
        d = 1
        while d < NBINS:
            v = v + jnp.concatenate(
                [jnp.zeros((NUM_CLASSES, d), jnp.float32), v[:, :-d]], axis=1)
            d *= 2
        return v

    cn = cumsum_last(n)
    cm = cumsum_last(n1)
    S = cm[:, NBINS - 1:NBINS]
    tot = cn[:, NBINS - 1:NBINS]
    ks = tot - cn
    ke = ks + n
    ss = S - cm
    se = ss + n1

    def jac(k, s):
        return jnp.where(k == 0.0, 0.0,
                         1.0 - (S - s) / jnp.maximum(S + k - s, 1.0))

    mid = (lax.broadcasted_iota(jnp.int32, (1, NBINS), 1).astype(jnp.float32)
           + 0.5) * jnp.float32(1.0 / NBINS)
    contrib = mid * (jac(ke, se) - jac(ks, ss))
    o_ref[...] = (jnp.sum(contrib) * jnp.float32(1.0 / NUM_CLASSES)
                  ).reshape(1, 1)


@jax.jit
def kernel(x, target):
    t32 = target.astype(jnp.int32)
    x4 = x.reshape(4, NUM_CLASSES, ROWS_PER_BATCH, 128)
    t4 = t32.reshape(4, ROWS_PER_BATCH, 128)

    idx = pl.pallas_call(
        _index_body,
        grid=(4, ROWS_PER_BATCH // ERR_BLOCK_ROWS),
        in_specs=[
            pl.BlockSpec((1, NUM_CLASSES, ERR_BLOCK_ROWS, 128),
                         lambda b, i: (b, 0, i, 0)),
            pl.BlockSpec((1, ERR_BLOCK_ROWS, 128), lambda b, i: (b, i, 0)),
        ],
        out_specs=pl.BlockSpec((1, NUM_CLASSES, ERR_BLOCK_ROWS, 128),
                               lambda b, i: (b, 0, i, 0)),
        out_shape=jax.ShapeDtypeStruct(
            (4, NUM_CLASSES, ROWS_PER_BATCH, 128), jnp.int32),
    )(x4, t4)
    idx_flat = idx.reshape(4 * NUM_CLASSES * SPATIAL)

    hist = pl.kernel(
        _sc_hist_body,
        out_type=jax.ShapeDtypeStruct(
            (NUM_CLASSES, NW, TBL), jnp.float32),
        mesh=plsc.VectorSubcoreMesh(core_axis_name="c", subcore_axis_name="s"),
        compiler_params=pltpu.CompilerParams(needs_layout_passes=False),
        scratch_types=[
            pltpu.VMEM((PER_W,), jnp.int32),
            pltpu.VMEM((LANES * TBL,), jnp.float32),
            pltpu.VMEM((TBL,), jnp.float32),
        ],
    )(idx_flat)

    loss = pl.pallas_call(
        _finalize_body,
        out_shape=jax.ShapeDtypeStruct((1, 1), jnp.float32),
    )(hist)
    return loss.reshape(())
